# Initial kernel scaffold; baseline (speedup 1.0000x reference)
import jax
import jax.numpy as jnp
from jax.experimental import pallas as pl


def kernel(input_ids, cu_seq_lens, cu_batch_size, embed_table):
    raise NotImplementedError("write your pallas kernel here")


# speedup vs baseline: 1.6059x; 1.6059x over previous
import functools

import jax
import jax.numpy as jnp
from jax import lax
from jax.experimental import pallas as pl
from jax.experimental.pallas import tpu as pltpu
from jax.experimental.pallas import tpu_sc as plsc

VOCAB = 32000
HIDDEN = 2048
BATCH = 2
SEQ = 4096
B = BATCH * SEQ
NC, NS = 2, 16
NW = NC * NS
BPW = B // NW
CHUNK = 16
NCHUNK = BPW // CHUNK


def _build_gather():
    mesh = plsc.VectorSubcoreMesh(core_axis_name="c", subcore_axis_name="s")

    @functools.partial(
        pl.kernel,
        mesh=mesh,
        out_type=jax.ShapeDtypeStruct((B, HIDDEN), jnp.float32),
        scratch_types=[
            pltpu.VMEM((BPW,), jnp.int32),
            pltpu.VMEM((CHUNK, HIDDEN), jnp.float32),
            pltpu.VMEM((CHUNK, HIDDEN), jnp.float32),
            pltpu.SemaphoreType.DMA,
            pltpu.SemaphoreType.DMA,
            pltpu.SemaphoreType.DMA,
            pltpu.SemaphoreType.DMA,
        ],
    )
    def gather_kernel(ids_hbm, table_hbm, out_hbm,
                      idx_v, buf0, buf1, g0, g1, p0, p1):
        wid = lax.axis_index("s") * NC + lax.axis_index("c")
        base = wid * BPW
        pltpu.sync_copy(ids_hbm.at[pl.ds(base, BPW)], idx_v)

        bufs = (buf0, buf1)
        gsems = (g0, g1)
        psems = (p0, p1)

        gathers = [None, None]
        puts = [None, None]
        for j in range(NCHUNK):
            b = j % 2
            if puts[b] is not None:
                puts[b].wait()
            cp = pltpu.make_async_copy(
                table_hbm.at[idx_v.at[pl.ds(j * CHUNK, CHUNK)]],
                bufs[b], gsems[b])
            cp.start()
            gathers[b] = cp
            if j >= 1:
                q = (j - 1) % 2
                gathers[q].wait()
                out_cp = pltpu.make_async_copy(
                    bufs[q], out_hbm.at[pl.ds(base + (j - 1) * CHUNK, CHUNK)],
                    psems[q])
                out_cp.start()
                puts[q] = out_cp
        last = (NCHUNK - 1) % 2
        gathers[last].wait()
        out_cp = pltpu.make_async_copy(
            bufs[last], out_hbm.at[pl.ds(base + (NCHUNK - 1) * CHUNK, CHUNK)],
            psems[last])
        out_cp.start()
        puts[last] = out_cp
        puts[0].wait()
        puts[1].wait()

    return gather_kernel


_gather = _build_gather()


def kernel(input_ids, cu_seq_lens, cu_batch_size, embed_table):
    ids_flat = input_ids.reshape(B).astype(jnp.int32)
    out = _gather(ids_flat, embed_table)
    return out.reshape(BATCH, SEQ, HIDDEN), cu_seq_lens, cu_batch_size
